# Initial kernel scaffold; baseline (speedup 1.0000x reference)
import functools

import jax
import jax.numpy as jnp
from jax import lax
from jax.experimental import pallas as pl
from jax.experimental.pallas import tpu as pltpu
from jax.experimental.pallas import tpu_sc as plsc

N = 10000
F_IN = 128
HID = 64
N_GRAPHS = 64
N_CLASSES = 10

NP = 10240
R = 1280
NBLK = NP // R

NC = 2
NS = 16
NW = NC * NS
K = 128
RPT = NP // NS

_MESH = dict(core_axis_name="c", subcore_axis_name="s")


def _sc_hist(dstp, ones_rows, zinit, nch):
    mesh = plsc.VectorSubcoreMesh(**_MESH)

    @functools.partial(
        pl.kernel,
        out_type=jax.ShapeDtypeStruct((NC, NP, HID), jnp.float32),
        mesh=mesh,
        scratch_types=[
            pltpu.VMEM((nch, K), jnp.int32),
            pltpu.VMEM((K, HID), jnp.float32),
            pltpu.VMEM_SHARED((NP, HID), jnp.float32),
        ],
    )
    def k(dst_hbm, ones_hbm, z_hbm, out_hbm, dstv, onesv, acc):
        c = lax.axis_index("c")
        s = lax.axis_index("s")
        wid = s * NC + c
        base = s * RPT
        pltpu.sync_copy(z_hbm.at[pl.ds(base, RPT)], acc.at[pl.ds(base, RPT)])
        pltpu.sync_copy(ones_hbm, onesv)
        pltpu.sync_copy(dst_hbm.at[wid], dstv)
        plsc.subcore_barrier()

        def step(j, carry):
            pltpu.sync_copy(onesv, acc.at[dstv.at[j]], add=True)
            return carry

        lax.fori_loop(0, nch, step, 0)
        plsc.subcore_barrier()
        pltpu.sync_copy(acc.at[pl.ds(base, RPT)],
                        out_hbm.at[c, pl.ds(base, RPT)])

    return k(dstp, ones_rows, zinit)


def _sc_msg(srcp, dstp, y, zinit, nch):
    mesh = plsc.VectorSubcoreMesh(**_MESH)

    @functools.partial(
        pl.kernel,
        out_type=jax.ShapeDtypeStruct((NC, NP, HID), jnp.float32),
        mesh=mesh,
        scratch_types=[
            pltpu.VMEM((nch, K), jnp.int32),
            pltpu.VMEM((nch, K), jnp.int32),
            pltpu.VMEM((K, HID), jnp.float32),
            pltpu.VMEM_SHARED((NP, HID), jnp.float32),
            pltpu.SemaphoreType.DMA,
        ],
    )
    def k(src_hbm, dst_hbm, y_hbm, z_hbm, out_hbm, srcv, dstv, rows, acc, sem):
        c = lax.axis_index("c")
        s = lax.axis_index("s")
        wid = s * NC + c
        base = s * RPT
        pltpu.sync_copy(z_hbm.at[pl.ds(base, RPT)], acc.at[pl.ds(base, RPT)])
        pltpu.sync_copy(src_hbm.at[wid], srcv)
        pltpu.sync_copy(dst_hbm.at[wid], dstv)
        plsc.subcore_barrier()

        def step(j, carry):
            pltpu.async_copy(y_hbm.at[srcv.at[j]], rows, sem).wait()
            pltpu.sync_copy(rows, acc.at[dstv.at[j]], add=True)
            return carry

        lax.fori_loop(0, nch, step, 0)
        plsc.subcore_barrier()
        pltpu.sync_copy(acc.at[pl.ds(base, RPT)],
                        out_hbm.at[c, pl.ds(base, RPT)])

    return k(srcp, dstp, y, zinit)


def _tc_scale1(x_pad, hist, W1):
    def body(x_ref, h_ref, w_ref, y_ref, dis_ref):
        i = pl.program_id(0)
        h = h_ref[...]
        deg = h[0] + h[1] + 1.0
        dis = lax.rsqrt(deg)
        row = lax.broadcasted_iota(jnp.int32, (R, HID), 0) + i * R
        dis = jnp.where(row < N, dis, 0.0)
        xw = jnp.dot(x_ref[...], w_ref[...], preferred_element_type=jnp.float32)
        y_ref[...] = dis * xw
        dis_ref[...] = dis

    return pl.pallas_call(
        body,
        grid=(NBLK,),
        in_specs=[
            pl.BlockSpec((R, F_IN), lambda i: (i, 0)),
            pl.BlockSpec((NC, R, HID), lambda i: (0, i, 0)),
            pl.BlockSpec((F_IN, HID), lambda i: (0, 0)),
        ],
        out_specs=[
            pl.BlockSpec((R, HID), lambda i: (i, 0)),
            pl.BlockSpec((R, HID), lambda i: (i, 0)),
        ],
        out_shape=[
            jax.ShapeDtypeStruct((NP, HID), jnp.float32),
            jax.ShapeDtypeStruct((NP, HID), jnp.float32),
        ],
    )(x_pad, hist, W1)


def _tc_layer2(acc1, y1, dis, b1r, W2):
    def body(a_ref, y1_ref, d_ref, b_ref, w_ref, y2_ref):
        a = a_ref[...]
        d = d_ref[...]
        o = d * (a[0] + a[1] + y1_ref[...]) + b_ref[...]
        h = jnp.maximum(o, 0.0)
        y2_ref[...] = d * jnp.dot(h, w_ref[...],
                                  preferred_element_type=jnp.float32)

    return pl.pallas_call(
        body,
        grid=(NBLK,),
        in_specs=[
            pl.BlockSpec((NC, R, HID), lambda i: (0, i, 0)),
            pl.BlockSpec((R, HID), lambda i: (i, 0)),
            pl.BlockSpec((R, HID), lambda i: (i, 0)),
            pl.BlockSpec((1, HID), lambda i: (0, 0)),
            pl.BlockSpec((HID, HID), lambda i: (0, 0)),
        ],
        out_specs=pl.BlockSpec((R, HID), lambda i: (i, 0)),
        out_shape=jax.ShapeDtypeStruct((NP, HID), jnp.float32),
    )(acc1, y1, dis, b1r, W2)


def _tc_head(acc2, y2, dis, b2r, batch2d, fcW1, fb1r, fcW2, fb2r):
    def body(a_ref, y2_ref, d_ref, b_ref, bt_ref, w1_ref, c1_ref, w2_ref,
             c2_ref, out_ref, gsum, cnt):
        i = pl.program_id(0)

        @pl.when(i == 0)
        def _():
            gsum[...] = jnp.zeros((N_GRAPHS, HID), jnp.float32)
            cnt[...] = jnp.zeros((N_GRAPHS, 1), jnp.float32)

        a = a_ref[...]
        d = d_ref[...]
        o = d * (a[0] + a[1] + y2_ref[...]) + b_ref[...]
        h2 = jnp.maximum(o, 0.0)
        bt = bt_ref[...]
        gid = lax.broadcasted_iota(jnp.int32, (N_GRAPHS, 1), 0)
        oh = (bt == gid).astype(jnp.float32)
        gsum[...] += jnp.dot(oh, h2, preferred_element_type=jnp.float32)
        cnt[...] += jnp.dot(oh, jnp.ones((R, 1), jnp.float32),
                            preferred_element_type=jnp.float32)

        @pl.when(i == NBLK - 1)
        def _():
            g = gsum[...] / jnp.maximum(cnt[...], 1.0)
            z = jnp.maximum(
                jnp.dot(g, w1_ref[...], preferred_element_type=jnp.float32)
                + c1_ref[...], 0.0)
            out_ref[...] = (jnp.dot(z, w2_ref[...],
                                    preferred_element_type=jnp.float32)
                            + c2_ref[...])

    return pl.pallas_call(
        body,
        grid=(NBLK,),
        in_specs=[
            pl.BlockSpec((NC, R, HID), lambda i: (0, i, 0)),
            pl.BlockSpec((R, HID), lambda i: (i, 0)),
            pl.BlockSpec((R, HID), lambda i: (i, 0)),
            pl.BlockSpec((1, HID), lambda i: (0, 0)),
            pl.BlockSpec((1, R), lambda i: (0, i)),
            pl.BlockSpec((HID, HID), lambda i: (0, 0)),
            pl.BlockSpec((1, HID), lambda i: (0, 0)),
            pl.BlockSpec((HID, N_CLASSES), lambda i: (0, 0)),
            pl.BlockSpec((1, N_CLASSES), lambda i: (0, 0)),
        ],
        out_specs=pl.BlockSpec((N_GRAPHS, N_CLASSES), lambda i: (0, 0)),
        out_shape=jax.ShapeDtypeStruct((N_GRAPHS, N_CLASSES), jnp.float32),
        scratch_shapes=[
            pltpu.VMEM((N_GRAPHS, HID), jnp.float32),
            pltpu.VMEM((N_GRAPHS, 1), jnp.float32),
        ],
        compiler_params=pltpu.CompilerParams(
            dimension_semantics=("arbitrary",)),
    )(acc2, y2, dis, b2r, batch2d, fcW1, fb1r, fcW2, fb2r)


def kernel(x, edge_index, batch, W1, b1, W2, b2, fcW1, fcb1, fcW2, fcb2):
    n, f_in = x.shape
    e = edge_index.shape[1]
    nch = -(-e // (NW * K))
    ep = NW * nch * K

    x_pad = jnp.zeros((NP, f_in), jnp.float32).at[:n].set(x)
    pad = jnp.full((ep - e,), n, jnp.int32)
    srcp = jnp.concatenate([edge_index[0], pad]).reshape(NW, nch, K)
    dstp = jnp.concatenate([edge_index[1], pad]).reshape(NW, nch, K)
    batch2d = jnp.concatenate(
        [batch, jnp.full((NP - n,), N_GRAPHS, jnp.int32)]).reshape(1, NP)
    ones_rows = jnp.ones((K, HID), jnp.float32)
    zinit = jnp.zeros((NP, HID), jnp.float32)
    b1r = b1.reshape(1, HID)
    b2r = b2.reshape(1, HID)
    fb1r = fcb1.reshape(1, HID)
    fb2r = fcb2.reshape(1, N_CLASSES)

    hist = _sc_hist(dstp, ones_rows, zinit, nch)
    y1, dis = _tc_scale1(x_pad, hist, W1)
    acc1 = _sc_msg(srcp, dstp, y1, zinit, nch)
    y2 = _tc_layer2(acc1, y1, dis, b1r, W2)
    acc2 = _sc_msg(srcp, dstp, y2, zinit, nch)
    return _tc_head(acc2, y2, dis, b2r, batch2d, fcW1, fb1r, fcW2, fb2r)


# speedup vs baseline: 11.0416x; 11.0416x over previous
import functools

import jax
import jax.numpy as jnp
from jax import lax
from jax.experimental import pallas as pl
from jax.experimental.pallas import tpu as pltpu
from jax.experimental.pallas import tpu_sc as plsc

N = 10000
F_IN = 128
HID = 64
HW = 128
N_GRAPHS = 64
N_CLASSES = 10

NP = 10240
R = 1280
NBLK = NP // R

NC = 2
NS = 16
NW = NC * NS
K = 128
RPT = NP // NS

_MESH = dict(core_axis_name="c", subcore_axis_name="s")


def _sc_hist(dstp, ones_rows, zinit, nch):
    mesh = plsc.VectorSubcoreMesh(**_MESH)

    @functools.partial(
        pl.kernel,
        out_type=jax.ShapeDtypeStruct((NC, NP, HW), jnp.float32),
        mesh=mesh,
        scratch_types=[
            pltpu.VMEM((nch, K), jnp.int32),
            pltpu.VMEM((K, HW), jnp.float32),
            pltpu.VMEM_SHARED((NP, HW), jnp.float32),
        ],
    )
    def k(dst_hbm, ones_hbm, z_hbm, out_hbm, dstv, onesv, acc):
        c = lax.axis_index("c")
        s = lax.axis_index("s")
        wid = s * NC + c
        base = s * RPT
        pltpu.sync_copy(z_hbm.at[pl.ds(base, RPT)], acc.at[pl.ds(base, RPT)])
        pltpu.sync_copy(ones_hbm, onesv)
        pltpu.sync_copy(dst_hbm.at[wid], dstv)
        plsc.subcore_barrier()

        def step(j, carry):
            pltpu.sync_copy(onesv, acc.at[dstv.at[j]], add=True)
            return carry

        lax.fori_loop(0, nch, step, 0)
        plsc.subcore_barrier()
        pltpu.sync_copy(acc.at[pl.ds(base, RPT)],
                        out_hbm.at[c, pl.ds(base, RPT)])

    return k(dstp, ones_rows, zinit)


def _sc_msg(srcp, dstp, y, zinit, nch):
    mesh = plsc.VectorSubcoreMesh(**_MESH)

    @functools.partial(
        pl.kernel,
        out_type=jax.ShapeDtypeStruct((NC, NP, HW), jnp.float32),
        mesh=mesh,
        scratch_types=[
            pltpu.VMEM((nch, K), jnp.int32),
            pltpu.VMEM((nch, K), jnp.int32),
            pltpu.VMEM((K, HW), jnp.float32),
            pltpu.VMEM_SHARED((NP, HW), jnp.float32),
            pltpu.SemaphoreType.DMA,
        ],
    )
    def k(src_hbm, dst_hbm, y_hbm, z_hbm, out_hbm, srcv, dstv, rows, acc, sem):
        c = lax.axis_index("c")
        s = lax.axis_index("s")
        wid = s * NC + c
        base = s * RPT
        pltpu.sync_copy(z_hbm.at[pl.ds(base, RPT)], acc.at[pl.ds(base, RPT)])
        pltpu.sync_copy(src_hbm.at[wid], srcv)
        pltpu.sync_copy(dst_hbm.at[wid], dstv)
        plsc.subcore_barrier()

        def step(j, carry):
            pltpu.async_copy(y_hbm.at[srcv.at[j]], rows, sem).wait()
            pltpu.sync_copy(rows, acc.at[dstv.at[j]], add=True)
            return carry

        lax.fori_loop(0, nch, step, 0)
        plsc.subcore_barrier()
        pltpu.sync_copy(acc.at[pl.ds(base, RPT)],
                        out_hbm.at[c, pl.ds(base, RPT)])

    return k(srcp, dstp, y, zinit)


def _tc_scale1(x_pad, hist, W1p):
    def body(x_ref, h_ref, w_ref, y_ref, dis_ref):
        i = pl.program_id(0)
        h = h_ref[...]
        deg = h[0] + h[1] + 1.0
        dis = lax.rsqrt(deg)
        row = lax.broadcasted_iota(jnp.int32, (R, HW), 0) + i * R
        dis = jnp.where(row < N, dis, 0.0)
        xw = jnp.dot(x_ref[...], w_ref[...], preferred_element_type=jnp.float32)
        y_ref[...] = dis * xw
        dis_ref[...] = dis

    return pl.pallas_call(
        body,
        grid=(NBLK,),
        in_specs=[
            pl.BlockSpec((R, F_IN), lambda i: (i, 0)),
            pl.BlockSpec((NC, R, HW), lambda i: (0, i, 0)),
            pl.BlockSpec((F_IN, HW), lambda i: (0, 0)),
        ],
        out_specs=[
            pl.BlockSpec((R, HW), lambda i: (i, 0)),
            pl.BlockSpec((R, HW), lambda i: (i, 0)),
        ],
        out_shape=[
            jax.ShapeDtypeStruct((NP, HW), jnp.float32),
            jax.ShapeDtypeStruct((NP, HW), jnp.float32),
        ],
    )(x_pad, hist, W1p)


def _tc_layer2(acc1, y1, dis, b1r, W2p):
    def body(a_ref, y1_ref, d_ref, b_ref, w_ref, y2_ref):
        a = a_ref[...]
        d = d_ref[...]
        o = d * (a[0] + a[1] + y1_ref[...]) + b_ref[...]
        h = jnp.maximum(o, 0.0)
        y2_ref[...] = d * jnp.dot(h, w_ref[...],
                                  preferred_element_type=jnp.float32)

    return pl.pallas_call(
        body,
        grid=(NBLK,),
        in_specs=[
            pl.BlockSpec((NC, R, HW), lambda i: (0, i, 0)),
            pl.BlockSpec((R, HW), lambda i: (i, 0)),
            pl.BlockSpec((R, HW), lambda i: (i, 0)),
            pl.BlockSpec((1, HW), lambda i: (0, 0)),
            pl.BlockSpec((HW, HW), lambda i: (0, 0)),
        ],
        out_specs=pl.BlockSpec((R, HW), lambda i: (i, 0)),
        out_shape=jax.ShapeDtypeStruct((NP, HW), jnp.float32),
    )(acc1, y1, dis, b1r, W2p)


def _tc_head(acc2, y2, dis, b2r, batch2d, fcW1p, fb1r, fcW2p, fb2r):
    def body(a_ref, y2_ref, d_ref, b_ref, bt_ref, w1_ref, c1_ref, w2_ref,
             c2_ref, out_ref, gsum, cnt):
        i = pl.program_id(0)

        @pl.when(i == 0)
        def _():
            gsum[...] = jnp.zeros((N_GRAPHS, HW), jnp.float32)
            cnt[...] = jnp.zeros((N_GRAPHS, 1), jnp.float32)

        a = a_ref[...]
        d = d_ref[...]
        o = d * (a[0] + a[1] + y2_ref[...]) + b_ref[...]
        h2 = jnp.maximum(o, 0.0)
        bt = bt_ref[...]
        gid = lax.broadcasted_iota(jnp.int32, (N_GRAPHS, 1), 0)
        oh = (bt == gid).astype(jnp.float32)
        gsum[...] += jnp.dot(oh, h2, preferred_element_type=jnp.float32)
        cnt[...] += jnp.dot(oh, jnp.ones((R, 1), jnp.float32),
                            preferred_element_type=jnp.float32)

        @pl.when(i == NBLK - 1)
        def _():
            g = gsum[...] / jnp.maximum(cnt[...], 1.0)
            z = jnp.maximum(
                jnp.dot(g, w1_ref[...], preferred_element_type=jnp.float32)
                + c1_ref[...], 0.0)
            out_ref[...] = (jnp.dot(z, w2_ref[...],
                                    preferred_element_type=jnp.float32)
                            + c2_ref[...])

    return pl.pallas_call(
        body,
        grid=(NBLK,),
        in_specs=[
            pl.BlockSpec((NC, R, HW), lambda i: (0, i, 0)),
            pl.BlockSpec((R, HW), lambda i: (i, 0)),
            pl.BlockSpec((R, HW), lambda i: (i, 0)),
            pl.BlockSpec((1, HW), lambda i: (0, 0)),
            pl.BlockSpec((1, R), lambda i: (0, i)),
            pl.BlockSpec((HW, HW), lambda i: (0, 0)),
            pl.BlockSpec((1, HW), lambda i: (0, 0)),
            pl.BlockSpec((HW, N_CLASSES), lambda i: (0, 0)),
            pl.BlockSpec((1, N_CLASSES), lambda i: (0, 0)),
        ],
        out_specs=pl.BlockSpec((N_GRAPHS, N_CLASSES), lambda i: (0, 0)),
        out_shape=jax.ShapeDtypeStruct((N_GRAPHS, N_CLASSES), jnp.float32),
        scratch_shapes=[
            pltpu.VMEM((N_GRAPHS, HW), jnp.float32),
            pltpu.VMEM((N_GRAPHS, 1), jnp.float32),
        ],
        compiler_params=pltpu.CompilerParams(
            dimension_semantics=("arbitrary",)),
    )(acc2, y2, dis, b2r, batch2d, fcW1p, fb1r, fcW2p, fb2r)


def kernel(x, edge_index, batch, W1, b1, W2, b2, fcW1, fcb1, fcW2, fcb2):
    n, f_in = x.shape
    e = edge_index.shape[1]
    nch = -(-e // (NW * K))
    ep = NW * nch * K

    x_pad = jnp.zeros((NP, f_in), jnp.float32).at[:n].set(x)
    pad = jnp.full((ep - e,), n, jnp.int32)
    srcp = jnp.concatenate([edge_index[0], pad]).reshape(NW, nch, K)
    dstp = jnp.concatenate([edge_index[1], pad]).reshape(NW, nch, K)
    batch2d = jnp.concatenate(
        [batch, jnp.full((NP - n,), N_GRAPHS, jnp.int32)]).reshape(1, NP)
    ones_rows = jnp.ones((K, HW), jnp.float32)
    zinit = jnp.zeros((NP, HW), jnp.float32)
    W1p = jnp.zeros((f_in, HW), jnp.float32).at[:, :HID].set(W1)
    W2p = jnp.zeros((HW, HW), jnp.float32).at[:HID, :HID].set(W2)
    fcW1p = jnp.zeros((HW, HW), jnp.float32).at[:HID, :HID].set(fcW1)
    fcW2p = jnp.zeros((HW, N_CLASSES), jnp.float32).at[:HID].set(fcW2)
    b1r = jnp.zeros((1, HW), jnp.float32).at[0, :HID].set(b1)
    b2r = jnp.zeros((1, HW), jnp.float32).at[0, :HID].set(b2)
    fb1r = jnp.zeros((1, HW), jnp.float32).at[0, :HID].set(fcb1)
    fb2r = fcb2.reshape(1, N_CLASSES)

    hist = _sc_hist(dstp, ones_rows, zinit, nch)
    y1, dis = _tc_scale1(x_pad, hist, W1p)
    acc1 = _sc_msg(srcp, dstp, y1, zinit, nch)
    y2 = _tc_layer2(acc1, y1, dis, b1r, W2p)
    acc2 = _sc_msg(srcp, dstp, y2, zinit, nch)
    return _tc_head(acc2, y2, dis, b2r, batch2d, fcW1p, fb1r, fcW2p, fb2r)
